# baseline (speedup 1.0000x reference)
import functools

import jax
import jax.numpy as jnp
from jax import lax
from jax.experimental import pallas as pl
from jax.experimental.pallas import tpu as pltpu
from jax.experimental.pallas import tpu_sc as plsc

D_MODEL = 128
BASE_YEAR = 2022
MAX_DELTA = 128
TABLE_ROWS = 2 * MAX_DELTA + 1
BATCH = 16384

NUM_CORES = 2
NUM_SUBCORES = 16
LANES = 16
NUM_WORKERS = NUM_CORES * NUM_SUBCORES
B_PER_W = BATCH // NUM_WORKERS
CHUNK = 64
N_CHUNKS = B_PER_W // CHUNK


def _make_kernel():
    mesh = plsc.VectorSubcoreMesh(
        core_axis_name="c", subcore_axis_name="s",
        num_cores=NUM_CORES, num_subcores=NUM_SUBCORES,
    )

    @functools.partial(
        pl.kernel,
        mesh=mesh,
        compiler_params=pltpu.CompilerParams(
            needs_layout_passes=False, skip_device_barrier=True),
        out_type=jax.ShapeDtypeStruct((BATCH, D_MODEL), jnp.float32),
        scratch_types=[
            pltpu.VMEM_SHARED((TABLE_ROWS, D_MODEL), jnp.float32),
            pltpu.VMEM((B_PER_W,), jnp.int32),
            pltpu.VMEM((N_CHUNKS, CHUNK), jnp.int32),
            pltpu.VMEM((B_PER_W, D_MODEL), jnp.float32),
            pltpu.SemaphoreType.DMA,
            pltpu.SemaphoreType.DMA,
            pltpu.SemaphoreType.DMA,
        ],
    )
    def k(years_hbm, pe_hbm, out_hbm, pe_sh, yrs_v, idx_v, rows_v,
          in_sem, gat_sem, out_sem):
        sid = lax.axis_index("s")
        wid = sid * NUM_CORES + lax.axis_index("c")
        base = wid * B_PER_W
        c_yr = pltpu.async_copy(years_hbm.at[pl.ds(base, B_PER_W)], yrs_v, in_sem)

        @pl.when(sid == 0)
        def _stage_table():
            pltpu.sync_copy(pe_hbm, pe_sh)

        c_yr.wait()
        plsc.subcore_barrier()

        per_chunk = CHUNK // LANES
        gathers = []
        for j in range(N_CHUNKS):
            for i in range(per_chunk):
                y = yrs_v[pl.ds((j * per_chunk + i) * LANES, LANES)]
                idx = jnp.clip(y - (BASE_YEAR - MAX_DELTA), 0, 2 * MAX_DELTA)
                idx_v[j, pl.ds(i * LANES, LANES)] = idx
            gathers.append(pltpu.async_copy(
                pe_sh.at[idx_v.at[j]],
                rows_v.at[pl.ds(j * CHUNK, CHUNK)],
                gat_sem,
            ))
        for j in range(N_CHUNKS):
            gathers[j].wait()
            pltpu.async_copy(
                rows_v.at[pl.ds(j * CHUNK, CHUNK)],
                out_hbm.at[pl.ds(base + j * CHUNK, CHUNK)],
                out_sem,
            )
        pltpu.make_async_copy(
            rows_v,
            out_hbm.at[pl.ds(base, B_PER_W)],
            out_sem,
        ).wait()

    return k


_gather = _make_kernel()


@jax.jit
def kernel(years, pe):
    return _gather(years.astype(jnp.int32), pe)


# speedup vs baseline: 1.0032x; 1.0032x over previous
import functools

import jax
import jax.numpy as jnp
from jax import lax
from jax.experimental import pallas as pl
from jax.experimental.pallas import tpu as pltpu
from jax.experimental.pallas import tpu_sc as plsc

D_MODEL = 128
BASE_YEAR = 2022
MAX_DELTA = 128
TABLE_ROWS = 2 * MAX_DELTA + 1
BATCH = 16384

NUM_CORES = 2
NUM_SUBCORES = 16
LANES = 16
NUM_WORKERS = NUM_CORES * NUM_SUBCORES
B_PER_W = BATCH // NUM_WORKERS
CHUNK = 64
N_CHUNKS = B_PER_W // CHUNK


def _make_kernel():
    mesh = plsc.VectorSubcoreMesh(
        core_axis_name="c", subcore_axis_name="s",
        num_cores=NUM_CORES, num_subcores=NUM_SUBCORES,
    )

    @functools.partial(
        pl.kernel,
        mesh=mesh,
        compiler_params=pltpu.CompilerParams(
            needs_layout_passes=False, skip_device_barrier=True,
            use_tc_tiling_on_sc=False),
        out_type=jax.ShapeDtypeStruct((BATCH, D_MODEL), jnp.float32),
        scratch_types=[
            pltpu.VMEM_SHARED((TABLE_ROWS, D_MODEL), jnp.float32),
            pltpu.VMEM((B_PER_W,), jnp.int32),
            pltpu.VMEM((N_CHUNKS, CHUNK), jnp.int32),
            pltpu.VMEM((B_PER_W, D_MODEL), jnp.float32),
            pltpu.SemaphoreType.DMA,
            pltpu.SemaphoreType.DMA,
            pltpu.SemaphoreType.DMA,
        ],
    )
    def k(years_hbm, pe_hbm, out_hbm, pe_sh, yrs_v, idx_v, rows_v,
          in_sem, gat_sem, out_sem):
        sid = lax.axis_index("s")
        wid = sid * NUM_CORES + lax.axis_index("c")
        base = wid * B_PER_W
        c_yr = pltpu.async_copy(years_hbm.at[pl.ds(base, B_PER_W)], yrs_v, in_sem)

        @pl.when(sid == 0)
        def _stage_table():
            pltpu.sync_copy(pe_hbm, pe_sh)

        c_yr.wait()
        plsc.subcore_barrier()

        per_chunk = CHUNK // LANES
        gathers = []
        for j in range(N_CHUNKS):
            for i in range(per_chunk):
                y = yrs_v[pl.ds((j * per_chunk + i) * LANES, LANES)]
                idx = jnp.clip(y - (BASE_YEAR - MAX_DELTA), 0, 2 * MAX_DELTA)
                idx_v[j, pl.ds(i * LANES, LANES)] = idx
            gathers.append(pltpu.async_copy(
                pe_sh.at[idx_v.at[j]],
                rows_v.at[pl.ds(j * CHUNK, CHUNK)],
                gat_sem,
            ))
        for j in range(N_CHUNKS):
            gathers[j].wait()
            pltpu.async_copy(
                rows_v.at[pl.ds(j * CHUNK, CHUNK)],
                out_hbm.at[pl.ds(base + j * CHUNK, CHUNK)],
                out_sem,
            )
        pltpu.make_async_copy(
            rows_v,
            out_hbm.at[pl.ds(base, B_PER_W)],
            out_sem,
        ).wait()

    return k


_gather = _make_kernel()


@jax.jit
def kernel(years, pe):
    return _gather(years.astype(jnp.int32), pe)
